# baseline (speedup 1.0000x reference)
import functools

import jax
import jax.numpy as jnp
from jax import lax
from jax.experimental import pallas as pl
from jax.experimental.pallas import tpu as pltpu
from jax.experimental.pallas import tpu_sc as plsc

EMB_ROWS = 1000000
EMB_DIM = 32
BATCH_SIZE = 16384

_LANES = 16
_ROWS_PER_SLAB = 128 // EMB_DIM
_NUM_CORES = 2
_NUM_SUBCORES = 16
_NUM_WORKERS = _NUM_CORES * _NUM_SUBCORES
_B_PER_W = BATCH_SIZE // _NUM_WORKERS
_CHUNK = 128
_NCHUNK = _B_PER_W // _CHUNK
_GROUPS_PER_CHUNK = _CHUNK // _LANES

_NSLAB = EMB_ROWS * EMB_DIM // 128
_FULL_COLS = EMB_ROWS // 128
_TAIL = EMB_ROWS - _FULL_COLS * 128
_MAX_T = -(-_FULL_COLS // _NUM_WORKERS)

_mesh = plsc.VectorSubcoreMesh(core_axis_name="c", subcore_axis_name="s")


@functools.partial(
    pl.kernel,
    mesh=_mesh,
    out_type=jax.ShapeDtypeStruct((_NSLAB, 128), jnp.float32),
    scratch_types=[
        pltpu.VMEM((2, EMB_DIM, 128), jnp.float32),
        pltpu.VMEM((2, EMB_DIM, 128), jnp.float32),
        pltpu.VMEM((EMB_DIM, _TAIL), jnp.float32),
        pltpu.VMEM((_TAIL // 4, 128), jnp.float32),
        pltpu.SemaphoreType.DMA,
        pltpu.SemaphoreType.DMA,
        pltpu.SemaphoreType.DMA,
        pltpu.SemaphoreType.DMA,
    ],
    compiler_params=pltpu.CompilerParams(needs_layout_passes=False),
)
def _transpose_kernel(wt_hbm, table_hbm, in_v, out_v, tin_v, tout_v,
                      isem0, isem1, osem0, osem1):
    isems = (isem0, isem1)
    osems = (osem0, osem1)
    wid = lax.axis_index("s") * _NUM_CORES + lax.axis_index("c")
    nt = (_FULL_COLS - wid + _NUM_WORKERS - 1) // _NUM_WORKERS
    lane = lax.iota(jnp.int32, _LANES)

    def col_of(t):
        return wid + t * _NUM_WORKERS

    def fire_in(t, par):
        return pltpu.async_copy(
            wt_hbm.at[:, pl.ds(col_of(t) * 128, 128)],
            in_v.at[par], isems[par])

    def fire_out(t, par):
        return pltpu.async_copy(
            out_v.at[par],
            table_hbm.at[pl.ds(col_of(t) * EMB_DIM, EMB_DIM)],
            osems[par])

    def shuffle(src, dst):
        for q in range(EMB_DIM):
            for g in range(128 // _LANES):
                l0 = g * _LANES
                col = jnp.full((_LANES,), 4 * q + l0 // EMB_DIM, jnp.int32)
                vals = plsc.load_gather(src, [lane + (l0 % EMB_DIM), col])
                dst[q, pl.ds(l0, _LANES)] = vals

    fire_in(0, 0)

    def pair(p, _):
        for par in (0, 1):
            t = 2 * p + par

            @pl.when(t < nt)
            def _body(t=t, par=par):
                @pl.when(t + 1 < nt)
                def _fire_next():
                    fire_in(t + 1, 1 - par)

                pltpu.make_async_copy(
                    wt_hbm.at[:, pl.ds(0, 128)], in_v.at[par],
                    isems[par]).wait()

                @pl.when(t >= 2)
                def _drain_out():
                    pltpu.make_async_copy(
                        out_v.at[par],
                        table_hbm.at[pl.ds(0, EMB_DIM)], osems[par]).wait()

                shuffle(in_v.at[par], out_v.at[par])
                fire_out(t, par)
        return _

    lax.fori_loop(0, (_MAX_T + 1) // 2, pair, 0)

    for par in (0, 1):
        pltpu.make_async_copy(
            out_v.at[par],
            table_hbm.at[pl.ds(0, EMB_DIM)], osems[par]).wait()

    @pl.when(wid == 0)
    def _tail():
        pltpu.sync_copy(wt_hbm.at[:, pl.ds(_FULL_COLS * 128, _TAIL)], tin_v)
        for q in range(_TAIL // 4):
            for g in range(128 // _LANES):
                l0 = g * _LANES
                col = jnp.full((_LANES,), 4 * q + l0 // EMB_DIM, jnp.int32)
                vals = plsc.load_gather(tin_v, [lane + (l0 % EMB_DIM), col])
                tout_v[q, pl.ds(l0, _LANES)] = vals
        pltpu.sync_copy(
            tout_v, table_hbm.at[pl.ds(_FULL_COLS * EMB_DIM, _TAIL // 4)])


@functools.partial(
    pl.kernel,
    mesh=_mesh,
    out_type=jax.ShapeDtypeStruct((BATCH_SIZE, EMB_DIM), jnp.float32),
    scratch_types=[
        pltpu.VMEM((_B_PER_W,), jnp.int32),
        pltpu.VMEM((_NCHUNK, _CHUNK), jnp.int32),
        pltpu.VMEM((2, _CHUNK, 128), jnp.float32),
        pltpu.VMEM((_B_PER_W, EMB_DIM), jnp.float32),
        pltpu.SemaphoreType.DMA,
        pltpu.SemaphoreType.DMA,
    ],
    compiler_params=pltpu.CompilerParams(needs_layout_passes=False),
)
def _gather_kernel(idx_hbm, table_hbm, out_hbm, idx_v, slab_v, rows_v, out_v,
                   sem0, sem1):
    sems = (sem0, sem1)
    wid = lax.axis_index("s") * _NUM_CORES + lax.axis_index("c")
    base = wid * _B_PER_W
    pltpu.sync_copy(idx_hbm.at[wid], idx_v)
    for t in range(_B_PER_W // _LANES):
        iv = idx_v[pl.ds(t * _LANES, _LANES)]
        j, o = divmod(t * _LANES, _CHUNK)
        slab_v[j, pl.ds(o, _LANES)] = iv >> 2

    def fire(j):
        return pltpu.async_copy(
            table_hbm.at[slab_v.at[j]], rows_v.at[j % 2], sems[j % 2])

    lane = lax.iota(jnp.int32, _LANES)
    copies = [None] * _NCHUNK
    copies[0] = fire(0)
    copies[1] = fire(1)
    for j in range(_NCHUNK):
        copies[j].wait()
        buf = rows_v.at[j % 2]

        def extract(t, _, j=j, buf=buf):
            j0 = j * _CHUNK + t * _LANES
            iv = idx_v[pl.ds(j0, _LANES)]
            col0 = (iv & (_ROWS_PER_SLAB - 1)) << 5
            local = lane + t * _LANES
            rows = lane + j0
            for c in range(EMB_DIM):
                vals = plsc.load_gather(buf, [local, col0 + c])
                plsc.store_scatter(
                    out_v, [rows, jnp.full((_LANES,), c, jnp.int32)], vals)
            return _

        lax.fori_loop(0, _GROUPS_PER_CHUNK, extract, 0)
        if j + 2 < _NCHUNK:
            copies[j + 2] = fire(j + 2)
    pltpu.sync_copy(out_v, out_hbm.at[pl.ds(base, _B_PER_W)])


def kernel(position, weight):
    idx = position.astype(jnp.int32).reshape(_NUM_WORKERS, _B_PER_W)
    table = _transpose_kernel(weight.T)
    return _gather_kernel(idx, table)


# speedup vs baseline: 1.5784x; 1.5784x over previous
import functools

import jax
import jax.numpy as jnp
from jax import lax
from jax.experimental import pallas as pl
from jax.experimental.pallas import tpu as pltpu
from jax.experimental.pallas import tpu_sc as plsc

EMB_DIM = 32
BATCH_SIZE = 16384

_NUM_CORES = 2
_NUM_SUBCORES = 16
_NUM_WORKERS = _NUM_CORES * _NUM_SUBCORES
_B_PER_W = BATCH_SIZE // _NUM_WORKERS
_CHUNK = 128
_NCHUNK = _B_PER_W // _CHUNK

_mesh = plsc.VectorSubcoreMesh(core_axis_name="c", subcore_axis_name="s")


@functools.partial(
    pl.kernel,
    mesh=_mesh,
    out_type=jax.ShapeDtypeStruct((BATCH_SIZE, EMB_DIM), jnp.float32),
    scratch_types=[
        pltpu.VMEM((_NCHUNK, _CHUNK), jnp.int32),
        pltpu.VMEM((_B_PER_W, EMB_DIM), jnp.float32),
        pltpu.SemaphoreType.DMA,
    ],
    compiler_params=pltpu.CompilerParams(use_tc_tiling_on_sc=False),
)
def _gather_kernel(idx_hbm, table_hbm, out_hbm, idx_v, rows_v, sem):
    wid = lax.axis_index("s") * _NUM_CORES + lax.axis_index("c")
    base = wid * _B_PER_W
    pltpu.sync_copy(idx_hbm.at[wid], idx_v)
    copies = []
    for j in range(_NCHUNK):
        copies.append(
            pltpu.async_copy(
                table_hbm.at[idx_v.at[j]],
                rows_v.at[pl.ds(j * _CHUNK, _CHUNK)],
                sem,
            )
        )
    for c in copies:
        c.wait()
    pltpu.sync_copy(rows_v, out_hbm.at[pl.ds(base, _B_PER_W)])


def kernel(position, weight):
    idx = position.astype(jnp.int32).reshape(_NUM_WORKERS, _NCHUNK, _CHUNK)
    return _gather_kernel(idx, weight)
